# baseline (speedup 1.0000x reference)
import functools

import numpy as np

import jax
import jax.numpy as jnp
from jax import lax
from jax.experimental import pallas as pl
from jax.experimental.pallas import tpu as pltpu
from jax.experimental.pallas import tpu_sc as plsc

ITEM_DIM = 64
ACTN_DIM = 32
OUT_DIM = ITEM_DIM + ACTN_DIM
PAD_DIM = 128
LANES = 16
HIST_PAD = 64
KB = 4
NBUF = 2
LAG = 1


def _make_kernel(batch: int, hist: int):
    info = plsc.get_sparse_core_info()
    nw = info.num_cores * info.num_subcores
    b_per_w = batch // nw
    n_units = b_per_w // KB
    unit_rows = KB * HIST_PAD

    mesh = plsc.VectorSubcoreMesh(core_axis_name="c", subcore_axis_name="s")

    @functools.partial(
        pl.kernel,
        mesh=mesh,
        compiler_params=pltpu.CompilerParams(needs_layout_passes=False),
        out_type=jax.ShapeDtypeStruct((batch * HIST_PAD, PAD_DIM), jnp.float32),
        scratch_types=[
            pltpu.VMEM((hist, b_per_w), jnp.int32),
            pltpu.VMEM((hist, b_per_w), jnp.int32),
            pltpu.VMEM((NBUF, KB, HIST_PAD), jnp.int32),
            pltpu.VMEM((NBUF, KB, HIST_PAD), jnp.int32),
            pltpu.VMEM((NBUF, unit_rows, PAD_DIM), jnp.float32),
            pltpu.VMEM((4, ACTN_DIM), jnp.float32),
            pltpu.SemaphoreType.DMA,
            pltpu.SemaphoreType.DMA,
        ],
    )
    def emb_kernel(idsT, aidsT, item_tab, actn_tab, out,
                   ids_blk, aids_blk, iidx_v, aidx_v, rows_v, atab_v,
                   gsem, osem):
        wid = lax.axis_index("s") * info.num_cores + lax.axis_index("c")
        b0 = wid * b_per_w
        w_base = wid * b_per_w * HIST_PAD

        pltpu.sync_copy(idsT.at[:, pl.ds(b0, b_per_w)], ids_blk)
        pltpu.sync_copy(aidsT.at[:, pl.ds(b0, b_per_w)], aids_blk)
        pltpu.sync_copy(actn_tab, atab_v)

        lane = lax.iota(jnp.int32, LANES)
        hp = jnp.int32(HIST_PAD)
        group_bb, group_off = [], []
        for g in range(unit_rows // LANES):
            r = g * LANES + lane
            bb = r // hp
            group_bb.append(bb)
            group_off.append(r - bb * hp)

        def build_idx(u, slot):
            loaded = []
            for bb in range(KB):
                b_v = jnp.full((LANES,), u * KB + bb, jnp.int32)
                for g in range(-(-HIST_PAD // LANES)):
                    l = g * LANES + lane
                    lm = jnp.where(l < hist, l, l - hist)
                    loaded.append((bb, g,
                                   plsc.load_gather(ids_blk, [lm, b_v]),
                                   plsc.load_gather(aids_blk, [lm, b_v])))
            for bb, g, ids, aids in loaded:
                if (g + 1) * LANES <= HIST_PAD:
                    iidx_v.at[slot, bb][pl.ds(g * LANES, LANES)] = ids
                    aidx_v.at[slot, bb][pl.ds(g * LANES, LANES)] = aids
                else:
                    m = lane < (HIST_PAD - g * LANES)
                    plsc.store_scatter(
                        iidx_v.at[slot, bb], [g * LANES + lane], ids, mask=m)
                    plsc.store_scatter(
                        aidx_v.at[slot, bb], [g * LANES + lane], aids, mask=m)

        def gathers(slot):
            return [
                pltpu.make_async_copy(
                    item_tab.at[iidx_v.at[slot, bb]],
                    rows_v.at[slot, pl.ds(bb * HIST_PAD, HIST_PAD)], gsem)
                for bb in range(KB)
            ]

        def fill_action(slot):
            aidx2 = aidx_v.at[slot]
            for g in range(unit_rows // LANES):
                aid = plsc.load_gather(aidx2, [group_bb[g], group_off[g]])
                row_idx = g * LANES + lane
                cols = [
                    plsc.load_gather(
                        atab_v, [aid, jnp.full((LANES,), j, jnp.int32)])
                    for j in range(ACTN_DIM)
                ]
                for j, col in enumerate(cols):
                    plsc.store_scatter(
                        rows_v.at[slot],
                        [row_idx, jnp.full((LANES,), ITEM_DIM + j, jnp.int32)],
                        col)

        def out_copy(u, slot):
            return pltpu.make_async_copy(
                rows_v.at[slot],
                out.at[pl.ds(w_base + u * unit_rows, unit_rows)], osem)

        def step(u, _):
            slot = lax.rem(u, NBUF)

            @pl.when(u < n_units)
            def _():
                @pl.when(u >= NBUF)
                def _():
                    out_copy(u - NBUF, slot).wait()
                build_idx(u, slot)
                for c in gathers(slot):
                    c.start()

            @pl.when(u >= LAG)
            def _():
                vslot = lax.rem(u - LAG, NBUF)
                for c in gathers(vslot):
                    c.wait()
                fill_action(vslot)
                out_copy(u - LAG, vslot).start()
            return ()

        lax.fori_loop(0, n_units + LAG, step, ())
        for t in range(NBUF):
            u = n_units - NBUF + t
            out_copy(u, lax.rem(jnp.int32(u), NBUF)).wait()

    return emb_kernel


def kernel(item_ids, action_ids, item_table, actn_table):
    b, l = item_ids.shape
    idsT = item_ids.astype(jnp.int32).T
    aidsT = action_ids.astype(jnp.int32).T
    tab128 = jnp.pad(item_table, ((0, 0), (0, PAD_DIM - ITEM_DIM)))
    out2d = _make_kernel(b, l)(idsT, aidsT, tab128, actn_table)
    return out2d.reshape(b, HIST_PAD, PAD_DIM)[:, :l, :OUT_DIM]


# speedup vs baseline: 1.2367x; 1.2367x over previous
import functools

import jax
import jax.numpy as jnp
from jax import lax
from jax.experimental import pallas as pl
from jax.experimental.pallas import tpu as pltpu
from jax.experimental.pallas import tpu_sc as plsc

ITEM_DIM = 64
ACTN_DIM = 32
OUT_DIM = ITEM_DIM + ACTN_DIM
PAD_DIM = 128
LANES = 16
HIST_PAD = 56
KB = 4
NBUF = 2
LAG = 1


def _make_kernel(batch: int, hist: int):
    info = plsc.get_sparse_core_info()
    nw = info.num_cores * info.num_subcores
    b_per_w = batch // nw
    n_units = b_per_w // KB
    unit_rows = KB * HIST_PAD

    mesh = plsc.VectorSubcoreMesh(core_axis_name="c", subcore_axis_name="s")

    @functools.partial(
        pl.kernel,
        mesh=mesh,
        compiler_params=pltpu.CompilerParams(needs_layout_passes=False),
        out_type=jax.ShapeDtypeStruct((batch * HIST_PAD, PAD_DIM), jnp.float32),
        scratch_types=[
            pltpu.VMEM((hist, b_per_w), jnp.int32),
            pltpu.VMEM((hist, b_per_w), jnp.int32),
            pltpu.VMEM((NBUF, KB, HIST_PAD), jnp.int32),
            pltpu.VMEM((NBUF, KB, HIST_PAD), jnp.int32),
            pltpu.VMEM((NBUF, unit_rows, PAD_DIM), jnp.float32),
            pltpu.VMEM((4, ACTN_DIM), jnp.float32),
            pltpu.SemaphoreType.DMA,
            pltpu.SemaphoreType.DMA,
        ],
    )
    def emb_kernel(idsT, aidsT, item_tab, actn_tab, out,
                   ids_blk, aids_blk, iidx_v, aidx_v, rows_v, atab_v,
                   gsem, osem):
        wid = lax.axis_index("s") * info.num_cores + lax.axis_index("c")
        b0 = wid * b_per_w
        w_base = wid * b_per_w * HIST_PAD

        pltpu.sync_copy(idsT.at[:, pl.ds(b0, b_per_w)], ids_blk)
        pltpu.sync_copy(aidsT.at[:, pl.ds(b0, b_per_w)], aids_blk)
        pltpu.sync_copy(actn_tab, atab_v)

        lane = lax.iota(jnp.int32, LANES)
        hp = jnp.int32(HIST_PAD)
        group_bb, group_off = [], []
        for g in range(unit_rows // LANES):
            r = g * LANES + lane
            bb = r // hp
            group_bb.append(bb)
            group_off.append(r - bb * hp)

        def build_idx(u, slot):
            loaded = []
            for bb in range(KB):
                b_v = jnp.full((LANES,), u * KB + bb, jnp.int32)
                for g in range(HIST_PAD // LANES + 1):
                    l = g * LANES + lane
                    lm = jnp.where(l < hist, l, l - hist)
                    loaded.append((bb, g,
                                   plsc.load_gather(ids_blk, [lm, b_v]),
                                   plsc.load_gather(aids_blk, [lm, b_v])))
            for bb, g, ids, aids in loaded:
                if (g + 1) * LANES <= HIST_PAD:
                    iidx_v.at[slot, bb][pl.ds(g * LANES, LANES)] = ids
                    aidx_v.at[slot, bb][pl.ds(g * LANES, LANES)] = aids
                else:
                    m = lane < (HIST_PAD - g * LANES)
                    plsc.store_scatter(
                        iidx_v.at[slot, bb], [g * LANES + lane], ids, mask=m)
                    plsc.store_scatter(
                        aidx_v.at[slot, bb], [g * LANES + lane], aids, mask=m)

        def gathers(slot):
            return [
                pltpu.make_async_copy(
                    item_tab.at[iidx_v.at[slot, bb]],
                    rows_v.at[slot, pl.ds(bb * HIST_PAD, HIST_PAD)], gsem)
                for bb in range(KB)
            ]

        def fill_action(slot):
            aidx2 = aidx_v.at[slot]
            for g in range(unit_rows // LANES):
                aid = plsc.load_gather(aidx2, [group_bb[g], group_off[g]])
                row_idx = g * LANES + lane
                cols = [
                    plsc.load_gather(
                        atab_v, [aid, jnp.full((LANES,), j, jnp.int32)])
                    for j in range(ACTN_DIM)
                ]
                for j, col in enumerate(cols):
                    plsc.store_scatter(
                        rows_v.at[slot],
                        [row_idx, jnp.full((LANES,), ITEM_DIM + j, jnp.int32)],
                        col)

        def out_copy(u, slot):
            return pltpu.make_async_copy(
                rows_v.at[slot],
                out.at[pl.ds(w_base + u * unit_rows, unit_rows)], osem)

        def step(u, _):
            slot = lax.rem(u, NBUF)

            @pl.when(u < n_units)
            def _():
                @pl.when(u >= NBUF)
                def _():
                    out_copy(u - NBUF, slot).wait()
                build_idx(u, slot)
                for c in gathers(slot):
                    c.start()

            @pl.when(u >= LAG)
            def _():
                vslot = lax.rem(u - LAG, NBUF)
                for c in gathers(vslot):
                    c.wait()
                fill_action(vslot)
                out_copy(u - LAG, vslot).start()
            return ()

        lax.fori_loop(0, n_units + LAG, step, ())
        for t in range(NBUF):
            u = n_units - NBUF + t
            out_copy(u, lax.rem(jnp.int32(u), NBUF)).wait()

    return emb_kernel


def kernel(item_ids, action_ids, item_table, actn_table):
    b, l = item_ids.shape
    idsT = item_ids.astype(jnp.int32).T
    aidsT = action_ids.astype(jnp.int32).T
    tab128 = jnp.pad(item_table, ((0, 0), (0, PAD_DIM - ITEM_DIM)))
    out2d = _make_kernel(b, l)(idsT, aidsT, tab128, actn_table)
    return out2d.reshape(b, HIST_PAD, PAD_DIM)[:, :l, :OUT_DIM]
